# Initial kernel scaffold; baseline (speedup 1.0000x reference)
import jax
import jax.numpy as jnp
from jax.experimental import pallas as pl


def kernel(z, codebook):
    raise NotImplementedError("write your pallas kernel here")


# speedup vs baseline: 1.0150x; 1.0150x over previous
import functools

import jax
import jax.numpy as jnp
from jax import lax
from jax.experimental import pallas as pl
from jax.experimental.pallas import tpu as pltpu
from jax.experimental.pallas import tpu_sc as plsc

NUM_CODES = 8192
CODE_DIM = 32
B_TOTAL = 65536
COMMITMENT_COST = 0.25

BLK_B = 256
NB = B_TOTAL // BLK_B

NC = 2
NS = 16
NW = NC * NS
BPW = B_TOTAL // NW
CHUNK = 128
NCHUNK = BPW // CHUNK


def _assign_body(z_ref, cbt_ref, codes_ref):
    z_blk = z_ref[...]
    cbt = cbt_ref[...]
    zc = jnp.dot(z_blk, cbt, preferred_element_type=jnp.float32)
    c2 = jnp.sum(cbt * cbt, axis=0, keepdims=True)
    z2 = jnp.sum(z_blk * z_blk, axis=1, keepdims=True)
    dists = jnp.sqrt(jnp.maximum(z2 + c2 - 2.0 * zc, 0.0))
    m = jnp.min(dists, axis=1, keepdims=True)
    ks = lax.broadcasted_iota(jnp.int32, dists.shape, 1)
    idx = jnp.min(jnp.where(dists == m, ks, NUM_CODES), axis=1)
    codes_ref[...] = idx.reshape(1, 1, BLK_B)


def _assign_codes(z, cbt):
    codes3 = pl.pallas_call(
        _assign_body,
        grid=(NB,),
        in_specs=[
            pl.BlockSpec((BLK_B, CODE_DIM), lambda i: (i, 0)),
            pl.BlockSpec((CODE_DIM, NUM_CODES), lambda i: (0, 0)),
        ],
        out_specs=pl.BlockSpec((1, 1, BLK_B), lambda i: (i, 0, 0)),
        out_shape=jax.ShapeDtypeStruct((NB, 1, BLK_B), jnp.int32),
    )(z, cbt)
    return codes3.reshape(B_TOTAL)


def _sc_finish_body(cb_hbm, codes_hbm, z_hbm, zqst_hbm, loss_hbm,
                    idx_v, rows_v, z_v, acc_v, sem):
    wid = lax.axis_index("s") * NC + lax.axis_index("c")
    base = wid * BPW
    pltpu.sync_copy(codes_hbm.at[pl.ds(wid * NCHUNK, NCHUNK), :], idx_v)

    def chunk(j, acc):
        off = base + j * CHUNK
        pltpu.async_copy(cb_hbm.at[idx_v.at[j]], rows_v, sem).wait()
        pltpu.sync_copy(z_hbm.at[pl.ds(off, CHUNK), :], z_v)

        def row(i, a):
            for h in range(CODE_DIM // 16):
                zz = z_v[i, pl.ds(h * 16, 16)]
                rr = rows_v[i, pl.ds(h * 16, 16)]
                dd = rr - zz
                z_v[i, pl.ds(h * 16, 16)] = zz + dd
                a = a + dd * dd
            return a

        acc = lax.fori_loop(0, CHUNK, row, acc)
        pltpu.sync_copy(z_v, zqst_hbm.at[pl.ds(off, CHUNK), :])
        return acc

    acc = lax.fori_loop(0, NCHUNK, chunk, jnp.zeros((16,), jnp.float32))
    acc_v[...] = acc
    pltpu.sync_copy(acc_v, loss_hbm.at[wid])


@functools.cache
def _sc_finish():
    return pl.kernel(
        _sc_finish_body,
        mesh=plsc.VectorSubcoreMesh(core_axis_name="c", subcore_axis_name="s"),
        out_type=[
            jax.ShapeDtypeStruct((B_TOTAL, CODE_DIM), jnp.float32),
            jax.ShapeDtypeStruct((NW, 16), jnp.float32),
        ],
        scratch_types=[
            pltpu.VMEM((NCHUNK, CHUNK), jnp.int32),
            pltpu.VMEM((CHUNK, CODE_DIM), jnp.float32),
            pltpu.VMEM((CHUNK, CODE_DIM), jnp.float32),
            pltpu.VMEM((16,), jnp.float32),
            pltpu.SemaphoreType.DMA,
        ],
        compiler_params=pltpu.CompilerParams(use_tc_tiling_on_sc=False),
    )


def kernel(z, codebook):
    cbt = codebook.T
    codes = _assign_codes(z, cbt)
    zqst, loss_parts = _sc_finish()(codebook, codes.reshape(-1, CHUNK), z)
    s = jnp.sum(loss_parts)
    n = jnp.float32(B_TOTAL * CODE_DIM)
    commitment_loss = s / n
    codebook_loss = s / n
    vq_loss = codebook_loss + COMMITMENT_COST * commitment_loss
    return (zqst, codes, commitment_loss, codebook_loss, vq_loss)


# speedup vs baseline: 1.7314x; 1.7058x over previous
import functools

import jax
import jax.numpy as jnp
from jax import lax
from jax.experimental import pallas as pl
from jax.experimental.pallas import tpu as pltpu
from jax.experimental.pallas import tpu_sc as plsc

NUM_CODES = 8192
CODE_DIM = 32
B_TOTAL = 65536
COMMITMENT_COST = 0.25

BLK_B = 256
NB = B_TOTAL // BLK_B

NC = 2
NS = 16
NW = NC * NS
BPW = B_TOTAL // NW
CHUNK = 128
NCHUNK = BPW // CHUNK


def _assign_body(z_ref, cbt_ref, codes_ref):
    z_blk = z_ref[...]
    cbt = cbt_ref[...]
    zc = jnp.dot(z_blk, cbt, preferred_element_type=jnp.float32)
    c2h = 0.5 * jnp.sum(cbt * cbt, axis=0, keepdims=True)
    score = c2h - zc
    m = jnp.min(score, axis=1, keepdims=True)
    ks = lax.broadcasted_iota(jnp.int32, score.shape, 1)
    idx = jnp.min(jnp.where(score == m, ks, NUM_CODES), axis=1)
    codes_ref[...] = idx.reshape(1, 1, BLK_B)


def _assign_codes(z, cbt):
    codes3 = pl.pallas_call(
        _assign_body,
        grid=(NB,),
        in_specs=[
            pl.BlockSpec((BLK_B, CODE_DIM), lambda i: (i, 0)),
            pl.BlockSpec((CODE_DIM, NUM_CODES), lambda i: (0, 0)),
        ],
        out_specs=pl.BlockSpec((1, 1, BLK_B), lambda i: (i, 0, 0)),
        out_shape=jax.ShapeDtypeStruct((NB, 1, BLK_B), jnp.int32),
    )(z, cbt)
    return codes3.reshape(B_TOTAL)


def _sc_finish_body(cb_hbm, codes_hbm, z_hbm, zqst_hbm, loss_hbm,
                    idx_v, rows_v, z_v, acc_v, sem):
    wid = lax.axis_index("s") * NC + lax.axis_index("c")
    base = wid * BPW
    pltpu.sync_copy(codes_hbm.at[pl.ds(wid * NCHUNK, NCHUNK), :], idx_v)

    def chunk(j, acc):
        off = base + j * CHUNK
        pltpu.async_copy(cb_hbm.at[idx_v.at[j]], rows_v, sem).wait()
        pltpu.sync_copy(z_hbm.at[pl.ds(off, CHUNK), :], z_v)

        def row(i, a):
            for h in range(CODE_DIM // 16):
                zz = z_v[i, pl.ds(h * 16, 16)]
                rr = rows_v[i, pl.ds(h * 16, 16)]
                dd = rr - zz
                z_v[i, pl.ds(h * 16, 16)] = zz + dd
                a = a + dd * dd
            return a

        acc = lax.fori_loop(0, CHUNK, row, acc)
        pltpu.sync_copy(z_v, zqst_hbm.at[pl.ds(off, CHUNK), :])
        return acc

    acc = lax.fori_loop(0, NCHUNK, chunk, jnp.zeros((16,), jnp.float32))
    acc_v[...] = acc
    pltpu.sync_copy(acc_v, loss_hbm.at[wid])


@functools.cache
def _sc_finish():
    return pl.kernel(
        _sc_finish_body,
        mesh=plsc.VectorSubcoreMesh(core_axis_name="c", subcore_axis_name="s"),
        out_type=[
            jax.ShapeDtypeStruct((B_TOTAL, CODE_DIM), jnp.float32),
            jax.ShapeDtypeStruct((NW, 16), jnp.float32),
        ],
        scratch_types=[
            pltpu.VMEM((NCHUNK, CHUNK), jnp.int32),
            pltpu.VMEM((CHUNK, CODE_DIM), jnp.float32),
            pltpu.VMEM((CHUNK, CODE_DIM), jnp.float32),
            pltpu.VMEM((16,), jnp.float32),
            pltpu.SemaphoreType.DMA,
        ],
        compiler_params=pltpu.CompilerParams(use_tc_tiling_on_sc=False),
    )


def kernel(z, codebook):
    cbt = codebook.T
    codes = _assign_codes(z, cbt)
    zqst, loss_parts = _sc_finish()(codebook, codes.reshape(-1, CHUNK), z)
    s = jnp.sum(loss_parts)
    n = jnp.float32(B_TOTAL * CODE_DIM)
    commitment_loss = s / n
    codebook_loss = s / n
    vq_loss = codebook_loss + COMMITMENT_COST * commitment_loss
    return (zqst, codes, commitment_loss, codebook_loss, vq_loss)


# speedup vs baseline: 1.7365x; 1.0030x over previous
import functools

import jax
import jax.numpy as jnp
from jax import lax
from jax.experimental import pallas as pl
from jax.experimental.pallas import tpu as pltpu
from jax.experimental.pallas import tpu_sc as plsc

NUM_CODES = 8192
CODE_DIM = 32
B_TOTAL = 65536
COMMITMENT_COST = 0.25

BLK_B = 512
NB = B_TOTAL // BLK_B

NC = 2
NS = 16
NW = NC * NS
BPW = B_TOTAL // NW
CHUNK = 128
NCHUNK = BPW // CHUNK


def _assign_body(z_ref, cbt_ref, codes_ref):
    z_blk = z_ref[...]
    cbt = cbt_ref[...]
    zc = jnp.dot(z_blk, cbt, preferred_element_type=jnp.float32)
    c2h = 0.5 * jnp.sum(cbt * cbt, axis=0, keepdims=True)
    score = c2h - zc
    m = jnp.min(score, axis=1, keepdims=True)
    ks = lax.broadcasted_iota(jnp.int32, score.shape, 1)
    idx = jnp.min(jnp.where(score == m, ks, NUM_CODES), axis=1)
    codes_ref[...] = idx.reshape(1, 1, BLK_B)


def _assign_codes(z, cbt):
    codes3 = pl.pallas_call(
        _assign_body,
        grid=(NB,),
        in_specs=[
            pl.BlockSpec((BLK_B, CODE_DIM), lambda i: (i, 0)),
            pl.BlockSpec((CODE_DIM, NUM_CODES), lambda i: (0, 0)),
        ],
        out_specs=pl.BlockSpec((1, 1, BLK_B), lambda i: (i, 0, 0)),
        out_shape=jax.ShapeDtypeStruct((NB, 1, BLK_B), jnp.int32),
        compiler_params=pltpu.CompilerParams(
            dimension_semantics=("arbitrary",)),
    )(z, cbt)
    return codes3.reshape(B_TOTAL)


def _sc_finish_body(cb_hbm, codes_hbm, z_hbm, zqst_hbm, loss_hbm,
                    idx_v, rows_v, z_v, acc_v, sem):
    wid = lax.axis_index("s") * NC + lax.axis_index("c")
    base = wid * BPW
    pltpu.sync_copy(codes_hbm.at[pl.ds(wid * NCHUNK, NCHUNK), :], idx_v)

    def chunk(j, acc):
        off = base + j * CHUNK
        pltpu.async_copy(cb_hbm.at[idx_v.at[j]], rows_v, sem).wait()
        pltpu.sync_copy(z_hbm.at[pl.ds(off, CHUNK), :], z_v)

        def row(i, a):
            for h in range(CODE_DIM // 16):
                zz = z_v[i, pl.ds(h * 16, 16)]
                rr = rows_v[i, pl.ds(h * 16, 16)]
                dd = rr - zz
                z_v[i, pl.ds(h * 16, 16)] = zz + dd
                a = a + dd * dd
            return a

        acc = lax.fori_loop(0, CHUNK, row, acc)
        pltpu.sync_copy(z_v, zqst_hbm.at[pl.ds(off, CHUNK), :])
        return acc

    acc = lax.fori_loop(0, NCHUNK, chunk, jnp.zeros((16,), jnp.float32))
    acc_v[...] = acc
    pltpu.sync_copy(acc_v, loss_hbm.at[wid])


@functools.cache
def _sc_finish():
    return pl.kernel(
        _sc_finish_body,
        mesh=plsc.VectorSubcoreMesh(core_axis_name="c", subcore_axis_name="s"),
        out_type=[
            jax.ShapeDtypeStruct((B_TOTAL, CODE_DIM), jnp.float32),
            jax.ShapeDtypeStruct((NW, 16), jnp.float32),
        ],
        scratch_types=[
            pltpu.VMEM((NCHUNK, CHUNK), jnp.int32),
            pltpu.VMEM((CHUNK, CODE_DIM), jnp.float32),
            pltpu.VMEM((CHUNK, CODE_DIM), jnp.float32),
            pltpu.VMEM((16,), jnp.float32),
            pltpu.SemaphoreType.DMA,
        ],
        compiler_params=pltpu.CompilerParams(use_tc_tiling_on_sc=False),
    )


def kernel(z, codebook):
    cbt = codebook.T
    codes = _assign_codes(z, cbt)
    zqst, loss_parts = _sc_finish()(codebook, codes.reshape(-1, CHUNK), z)
    s = jnp.sum(loss_parts)
    n = jnp.float32(B_TOTAL * CODE_DIM)
    commitment_loss = s / n
    codebook_loss = s / n
    vq_loss = codebook_loss + COMMITMENT_COST * commitment_loss
    return (zqst, codes, commitment_loss, codebook_loss, vq_loss)


# speedup vs baseline: 1.7440x; 1.0043x over previous
import functools

import jax
import jax.numpy as jnp
from jax import lax
from jax.experimental import pallas as pl
from jax.experimental.pallas import tpu as pltpu
from jax.experimental.pallas import tpu_sc as plsc

NUM_CODES = 8192
CODE_DIM = 32
B_TOTAL = 65536
COMMITMENT_COST = 0.25

BLK_B = 512
NB = B_TOTAL // BLK_B

NC = 2
NS = 16
NW = NC * NS
BPW = B_TOTAL // NW
CHUNK = 128
NCHUNK = BPW // CHUNK


def _assign_body(z_ref, cbt_ref, codes_ref):
    z_blk = z_ref[...]
    cbt = cbt_ref[...]
    zc = jnp.dot(z_blk, cbt, preferred_element_type=jnp.float32)
    c2h = 0.5 * jnp.sum(cbt * cbt, axis=0, keepdims=True)
    score = c2h - zc
    u = lax.bitcast_convert_type(score, jnp.int32)
    s = u ^ (lax.shift_right_arithmetic(u, 31) & jnp.int32(0x7FFFFFFF))
    ks = lax.broadcasted_iota(jnp.int32, score.shape, 1)
    p = (s & jnp.int32(-NUM_CODES)) | ks
    idx = jnp.min(p, axis=1) & jnp.int32(NUM_CODES - 1)
    codes_ref[...] = idx.reshape(1, 1, BLK_B)


def _assign_codes(z, cbt):
    codes3 = pl.pallas_call(
        _assign_body,
        grid=(NB,),
        in_specs=[
            pl.BlockSpec((BLK_B, CODE_DIM), lambda i: (i, 0)),
            pl.BlockSpec((CODE_DIM, NUM_CODES), lambda i: (0, 0)),
        ],
        out_specs=pl.BlockSpec((1, 1, BLK_B), lambda i: (i, 0, 0)),
        out_shape=jax.ShapeDtypeStruct((NB, 1, BLK_B), jnp.int32),
        compiler_params=pltpu.CompilerParams(
            dimension_semantics=("arbitrary",)),
    )(z, cbt)
    return codes3.reshape(B_TOTAL)


def _sc_finish_body(cb_hbm, codes_hbm, z_hbm, zqst_hbm, loss_hbm,
                    idx_v, rows_v, z_v, acc_v, sem):
    wid = lax.axis_index("s") * NC + lax.axis_index("c")
    base = wid * BPW
    pltpu.sync_copy(codes_hbm.at[pl.ds(wid * NCHUNK, NCHUNK), :], idx_v)

    def chunk(j, acc):
        off = base + j * CHUNK
        pltpu.async_copy(cb_hbm.at[idx_v.at[j]], rows_v, sem).wait()
        pltpu.sync_copy(z_hbm.at[pl.ds(off, CHUNK), :], z_v)

        def row(i, a):
            for h in range(CODE_DIM // 16):
                zz = z_v[i, pl.ds(h * 16, 16)]
                rr = rows_v[i, pl.ds(h * 16, 16)]
                dd = rr - zz
                z_v[i, pl.ds(h * 16, 16)] = zz + dd
                a = a + dd * dd
            return a

        acc = lax.fori_loop(0, CHUNK, row, acc)
        pltpu.sync_copy(z_v, zqst_hbm.at[pl.ds(off, CHUNK), :])
        return acc

    acc = lax.fori_loop(0, NCHUNK, chunk, jnp.zeros((16,), jnp.float32))
    acc_v[...] = acc
    pltpu.sync_copy(acc_v, loss_hbm.at[wid])


@functools.cache
def _sc_finish():
    return pl.kernel(
        _sc_finish_body,
        mesh=plsc.VectorSubcoreMesh(core_axis_name="c", subcore_axis_name="s"),
        out_type=[
            jax.ShapeDtypeStruct((B_TOTAL, CODE_DIM), jnp.float32),
            jax.ShapeDtypeStruct((NW, 16), jnp.float32),
        ],
        scratch_types=[
            pltpu.VMEM((NCHUNK, CHUNK), jnp.int32),
            pltpu.VMEM((CHUNK, CODE_DIM), jnp.float32),
            pltpu.VMEM((CHUNK, CODE_DIM), jnp.float32),
            pltpu.VMEM((16,), jnp.float32),
            pltpu.SemaphoreType.DMA,
        ],
        compiler_params=pltpu.CompilerParams(use_tc_tiling_on_sc=False),
    )


def kernel(z, codebook):
    cbt = codebook.T
    codes = _assign_codes(z, cbt)
    zqst, loss_parts = _sc_finish()(codebook, codes.reshape(-1, CHUNK), z)
    s = jnp.sum(loss_parts)
    n = jnp.float32(B_TOTAL * CODE_DIM)
    commitment_loss = s / n
    codebook_loss = s / n
    vq_loss = codebook_loss + COMMITMENT_COST * commitment_loss
    return (zqst, codes, commitment_loss, codebook_loss, vq_loss)


# speedup vs baseline: 1.8944x; 1.0862x over previous
import functools

import jax
import jax.numpy as jnp
from jax import lax
from jax.experimental import pallas as pl
from jax.experimental.pallas import tpu as pltpu
from jax.experimental.pallas import tpu_sc as plsc

NUM_CODES = 8192
CODE_DIM = 32
B_TOTAL = 65536
COMMITMENT_COST = 0.25

AUG = 40
BLK_B = 512
NB = B_TOTAL // BLK_B

NC = 2
NS = 16
NW = NC * NS
BPW = B_TOTAL // NW
CHUNK = 128
NCHUNK = BPW // CHUNK


def _prep_body(cbt_ref, w_ref):
    cbt = cbt_ref[...]
    c2h = 0.5 * jnp.sum(cbt * cbt, axis=0, keepdims=True)
    pad = jnp.zeros((AUG - CODE_DIM - 1, NUM_CODES), jnp.float32)
    w_ref[...] = jnp.concatenate([-cbt, c2h, pad], axis=0)


def _assign_body(z_ref, w_ref, ks_ref, codes_ref):
    z_aug = z_ref[...]
    w = w_ref[...]
    score = jnp.dot(z_aug, w, preferred_element_type=jnp.float32)
    u = lax.bitcast_convert_type(score, jnp.int32)
    s = u ^ (lax.shift_right_arithmetic(u, 31) & jnp.int32(0x7FFFFFFF))
    p = (s & jnp.int32(-NUM_CODES)) | ks_ref[0:1, :]
    idx = jnp.min(p, axis=1) & jnp.int32(NUM_CODES - 1)
    codes_ref[...] = idx.reshape(1, 1, BLK_B)


def _assign_codes(z_aug, cbt):
    w_aug = pl.pallas_call(
        _prep_body,
        out_shape=jax.ShapeDtypeStruct((AUG, NUM_CODES), jnp.float32),
    )(cbt)
    ks = jax.lax.broadcasted_iota(jnp.int32, (8, NUM_CODES), 1)
    codes3 = pl.pallas_call(
        _assign_body,
        grid=(NB,),
        in_specs=[
            pl.BlockSpec((BLK_B, AUG), lambda i: (i, 0)),
            pl.BlockSpec((AUG, NUM_CODES), lambda i: (0, 0)),
            pl.BlockSpec((8, NUM_CODES), lambda i: (0, 0)),
        ],
        out_specs=pl.BlockSpec((1, 1, BLK_B), lambda i: (i, 0, 0)),
        out_shape=jax.ShapeDtypeStruct((NB, 1, BLK_B), jnp.int32),
        compiler_params=pltpu.CompilerParams(
            dimension_semantics=("arbitrary",)),
    )(z_aug, w_aug, ks)
    return codes3.reshape(B_TOTAL)


def _sc_finish_body(cb_hbm, codes_hbm, z_hbm, zqst_hbm, loss_hbm,
                    idx_v, rows_v, z_v, acc_v, sem):
    wid = lax.axis_index("s") * NC + lax.axis_index("c")
    base = wid * BPW
    pltpu.sync_copy(codes_hbm.at[pl.ds(wid * NCHUNK, NCHUNK), :], idx_v)

    def chunk(j, acc):
        off = base + j * CHUNK
        pltpu.async_copy(cb_hbm.at[idx_v.at[j]], rows_v, sem).wait()
        pltpu.sync_copy(z_hbm.at[pl.ds(off, CHUNK), :], z_v)

        def row(i, a):
            for h in range(CODE_DIM // 16):
                zz = z_v[i, pl.ds(h * 16, 16)]
                rr = rows_v[i, pl.ds(h * 16, 16)]
                dd = rr - zz
                z_v[i, pl.ds(h * 16, 16)] = zz + dd
                a = a + dd * dd
            return a

        acc = lax.fori_loop(0, CHUNK, row, acc)
        pltpu.sync_copy(z_v, zqst_hbm.at[pl.ds(off, CHUNK), :])
        return acc

    acc = lax.fori_loop(0, NCHUNK, chunk, jnp.zeros((16,), jnp.float32))
    acc_v[...] = acc
    pltpu.sync_copy(acc_v, loss_hbm.at[wid])


@functools.cache
def _sc_finish():
    return pl.kernel(
        _sc_finish_body,
        mesh=plsc.VectorSubcoreMesh(core_axis_name="c", subcore_axis_name="s"),
        out_type=[
            jax.ShapeDtypeStruct((B_TOTAL, CODE_DIM), jnp.float32),
            jax.ShapeDtypeStruct((NW, 16), jnp.float32),
        ],
        scratch_types=[
            pltpu.VMEM((NCHUNK, CHUNK), jnp.int32),
            pltpu.VMEM((CHUNK, CODE_DIM), jnp.float32),
            pltpu.VMEM((CHUNK, CODE_DIM), jnp.float32),
            pltpu.VMEM((16,), jnp.float32),
            pltpu.SemaphoreType.DMA,
        ],
        compiler_params=pltpu.CompilerParams(use_tc_tiling_on_sc=False),
    )


def kernel(z, codebook):
    cbt = codebook.T
    ones = jnp.ones((B_TOTAL, 1), jnp.float32)
    zpad = jnp.zeros((B_TOTAL, AUG - CODE_DIM - 1), jnp.float32)
    z_aug = jnp.concatenate([z, ones, zpad], axis=1)
    codes = _assign_codes(z_aug, cbt)
    zqst, loss_parts = _sc_finish()(codebook, codes.reshape(-1, CHUNK), z)
    s = jnp.sum(loss_parts)
    n = jnp.float32(B_TOTAL * CODE_DIM)
    commitment_loss = s / n
    codebook_loss = s / n
    vq_loss = codebook_loss + COMMITMENT_COST * commitment_loss
    return (zqst, codes, commitment_loss, codebook_loss, vq_loss)


# speedup vs baseline: 2.6615x; 1.4049x over previous
import functools

import jax
import jax.numpy as jnp
from jax import lax
from jax.experimental import pallas as pl
from jax.experimental.pallas import tpu as pltpu
from jax.experimental.pallas import tpu_sc as plsc

NUM_CODES = 8192
CODE_DIM = 32
B_TOTAL = 65536
COMMITMENT_COST = 0.25

AUG = 40
BLK_B = 512
NB = B_TOTAL // BLK_B

NC = 2
NS = 16
NW = NC * NS
BPW = B_TOTAL // NW
CHUNK = 128
NCHUNK = BPW // CHUNK


def _prep_body(cbt_ref, w_ref):
    cbt = cbt_ref[...]
    c2h = 0.5 * jnp.sum(cbt * cbt, axis=0, keepdims=True)
    pad = jnp.zeros((AUG - CODE_DIM - 1, NUM_CODES), jnp.float32)
    w_ref[...] = jnp.concatenate([-cbt, c2h, pad], axis=0)


def _assign_body(z_ref, w_ref, ks_ref, codes_ref):
    z_aug = z_ref[...]
    w = w_ref[...]
    score = jnp.dot(z_aug, w, preferred_element_type=jnp.float32)
    u = lax.bitcast_convert_type(score, jnp.int32)
    p = lax.bitcast_convert_type(
        (u & jnp.int32(-NUM_CODES)) | ks_ref[0:1, :], jnp.float32)
    m = jnp.min(p, axis=1)
    idx = lax.bitcast_convert_type(m, jnp.int32) & jnp.int32(NUM_CODES - 1)
    codes_ref[...] = idx.reshape(1, 1, BLK_B)


def _assign_codes(z_aug, cbt):
    w_aug = pl.pallas_call(
        _prep_body,
        out_shape=jax.ShapeDtypeStruct((AUG, NUM_CODES), jnp.float32),
    )(cbt)
    ks = jax.lax.broadcasted_iota(jnp.int32, (8, NUM_CODES), 1)
    codes3 = pl.pallas_call(
        _assign_body,
        grid=(NB,),
        in_specs=[
            pl.BlockSpec((BLK_B, AUG), lambda i: (i, 0)),
            pl.BlockSpec((AUG, NUM_CODES), lambda i: (0, 0)),
            pl.BlockSpec((8, NUM_CODES), lambda i: (0, 0)),
        ],
        out_specs=pl.BlockSpec((1, 1, BLK_B), lambda i: (i, 0, 0)),
        out_shape=jax.ShapeDtypeStruct((NB, 1, BLK_B), jnp.int32),
        compiler_params=pltpu.CompilerParams(
            dimension_semantics=("arbitrary",)),
    )(z_aug, w_aug, ks)
    return codes3.reshape(B_TOTAL)


def _sc_finish_body(cb_hbm, codes_hbm, z_hbm, zqst_hbm, loss_hbm,
                    idx_v, rows_v, z_v, acc_v, sem):
    wid = lax.axis_index("s") * NC + lax.axis_index("c")
    base = wid * BPW
    pltpu.sync_copy(codes_hbm.at[pl.ds(wid * NCHUNK, NCHUNK), :], idx_v)

    def chunk(j, acc):
        off = base + j * CHUNK
        pltpu.async_copy(cb_hbm.at[idx_v.at[j]], rows_v, sem).wait()
        pltpu.sync_copy(z_hbm.at[pl.ds(off, CHUNK), :], z_v)

        def row(i, a):
            for h in range(CODE_DIM // 16):
                zz = z_v[i, pl.ds(h * 16, 16)]
                rr = rows_v[i, pl.ds(h * 16, 16)]
                dd = rr - zz
                z_v[i, pl.ds(h * 16, 16)] = zz + dd
                a = a + dd * dd
            return a

        acc = lax.fori_loop(0, CHUNK, row, acc)
        pltpu.sync_copy(z_v, zqst_hbm.at[pl.ds(off, CHUNK), :])
        return acc

    acc = lax.fori_loop(0, NCHUNK, chunk, jnp.zeros((16,), jnp.float32))
    acc_v[...] = acc
    pltpu.sync_copy(acc_v, loss_hbm.at[wid])


@functools.cache
def _sc_finish():
    return pl.kernel(
        _sc_finish_body,
        mesh=plsc.VectorSubcoreMesh(core_axis_name="c", subcore_axis_name="s"),
        out_type=[
            jax.ShapeDtypeStruct((B_TOTAL, CODE_DIM), jnp.float32),
            jax.ShapeDtypeStruct((NW, 16), jnp.float32),
        ],
        scratch_types=[
            pltpu.VMEM((NCHUNK, CHUNK), jnp.int32),
            pltpu.VMEM((CHUNK, CODE_DIM), jnp.float32),
            pltpu.VMEM((CHUNK, CODE_DIM), jnp.float32),
            pltpu.VMEM((16,), jnp.float32),
            pltpu.SemaphoreType.DMA,
        ],
        compiler_params=pltpu.CompilerParams(use_tc_tiling_on_sc=False),
    )


def kernel(z, codebook):
    cbt = codebook.T
    ones = jnp.ones((B_TOTAL, 1), jnp.float32)
    zpad = jnp.zeros((B_TOTAL, AUG - CODE_DIM - 1), jnp.float32)
    z_aug = jnp.concatenate([z, ones, zpad], axis=1)
    codes = _assign_codes(z_aug, cbt)
    zqst, loss_parts = _sc_finish()(codebook, codes.reshape(-1, CHUNK), z)
    s = jnp.sum(loss_parts)
    n = jnp.float32(B_TOTAL * CODE_DIM)
    commitment_loss = s / n
    codebook_loss = s / n
    vq_loss = codebook_loss + COMMITMENT_COST * commitment_loss
    return (zqst, codes, commitment_loss, codebook_loss, vq_loss)
